# baseline (speedup 1.0000x reference)
import functools

import jax
import jax.numpy as jnp
from jax import lax
from jax.experimental import pallas as pl
from jax.experimental.pallas import tpu as pltpu
from jax.experimental.pallas import tpu_sc as plsc

_P = 64


def _positions_kernel(parts_ref, inv_ref, p0_ref, p1_ref, cs_ref, *, n):
    b = pl.program_id(0)
    p2 = parts_ref[0]
    R, C = p2.shape
    Q = R * _P

    f32 = jnp.float32
    hi = lax.Precision.HIGHEST

    qi_qc = lax.broadcasted_iota(jnp.int32, (Q, C), 0)
    piot = (qi_qc % _P)
    oh = (lax.broadcasted_iota(jnp.int32, (Q, _P), 0) % _P
          == lax.broadcasted_iota(jnp.int32, (Q, _P), 1)).astype(f32)
    ci = lax.broadcasted_iota(jnp.int32, (C, C), 0)
    cj = lax.broadcasted_iota(jnp.int32, (C, C), 1)
    u_strict = (ci < cj).astype(f32)
    l_strict = (cj < ci).astype(f32)
    p_i = lax.broadcasted_iota(jnp.int32, (_P, _P), 0)
    p_j = lax.broadcasted_iota(jnp.int32, (_P, _P), 1)
    up_strict = (p_i < p_j).astype(f32)

    rep = jnp.repeat(p2.astype(f32), _P, axis=0)
    ew = (rep == piot.astype(f32)).astype(f32)

    prefw = jnp.dot(ew, u_strict, preferred_element_type=f32)

    rc = jnp.sum(ew, axis=1, keepdims=True)
    rc2 = jnp.sum((rc * oh).reshape(R, _P, _P), axis=1)

    before2 = jnp.dot(l_strict[:, :R], rc2, preferred_element_type=f32)
    counts = jnp.sum(rc2, axis=0, keepdims=True)
    base = jnp.dot(counts, up_strict, preferred_element_type=f32,
                   precision=hi)
    bb2 = before2 + base

    bbrep = jnp.repeat(bb2, _P, axis=0)
    bbflat = jnp.sum(bbrep * oh, axis=1, keepdims=True)

    pos = jnp.sum((ew * (prefw + bbflat)).reshape(R, _P, C), axis=1)

    incl = counts + base
    posg = pos.astype(jnp.int32) + b * n
    inv_ref[0] = posg
    pos_pair = posg.reshape(R, C // 2, 2)
    p0_ref[0] = pos_pair[:, :, 0]
    p1_ref[0] = pos_pair[:, :, 1]
    cs_ref[0] = incl.astype(jnp.int32) + b * n


def _compute_positions(parts3):
    B = parts3.shape[0]
    n = parts3.shape[1] * parts3.shape[2]
    return pl.pallas_call(
        functools.partial(_positions_kernel, n=n),
        grid=(B,),
        in_specs=[pl.BlockSpec((1, 128, 128), lambda b: (b, 0, 0))],
        out_specs=[
            pl.BlockSpec((1, 128, 128), lambda b: (b, 0, 0)),
            pl.BlockSpec((1, 128, 64), lambda b: (b, 0, 0)),
            pl.BlockSpec((1, 128, 64), lambda b: (b, 0, 0)),
            pl.BlockSpec((1, 1, _P), lambda b: (b, 0, 0)),
        ],
        out_shape=[
            jax.ShapeDtypeStruct((B, 128, 128), jnp.int32),
            jax.ShapeDtypeStruct((B, 128, 64), jnp.int32),
            jax.ShapeDtypeStruct((B, 128, 64), jnp.int32),
            jax.ShapeDtypeStruct((B, 1, _P), jnp.int32),
        ],
    )(parts3)


_CH = 32


_NBUF = 3


def _scatter_body(x_hbm, pos0_hbm, pos1_hbm, out_hbm,
                  rows_v, idx0_v, idx1_v, l0, l1, l2, s0, s1, s2,
                  *, rows_per_w, nc):
    wid = lax.axis_index("s") * nc + lax.axis_index("c")
    w0 = wid * rows_per_w
    n_chunks = rows_per_w // _CH
    lsem = (l0, l1, l2)
    ssem = (s0, s1, s2)

    def start_load(c):
        b = c % _NBUF
        sl = pl.ds(w0 + c * _CH, _CH)
        return (
            pltpu.async_copy(pos0_hbm.at[sl], idx0_v.at[b], lsem[b]),
            pltpu.async_copy(pos1_hbm.at[sl], idx1_v.at[b], lsem[b]),
            pltpu.async_copy(x_hbm.at[sl], rows_v.at[b], lsem[b]),
        )

    def start_scatter(c):
        b = c % _NBUF
        return (
            pltpu.async_copy(rows_v.at[b], out_hbm.at[idx0_v.at[b]], ssem[b]),
            pltpu.async_copy(rows_v.at[b], out_hbm.at[idx1_v.at[b]], ssem[b]),
        )

    loads = {0: start_load(0), 1: start_load(1)}
    scats = {}
    for c in range(n_chunks):
        for h in loads.pop(c):
            h.wait()
        scats[c] = start_scatter(c)
        nxt = c + 2
        if nxt < n_chunks:
            if nxt - _NBUF in scats:
                for h in scats.pop(nxt - _NBUF):
                    h.wait()
            loads[nxt] = start_load(nxt)
    for hs in scats.values():
        for h in hs:
            h.wait()


def _scatter_rows(x2, pos0, pos1):
    N, d = x2.shape
    info = plsc.get_sparse_core_info()
    nc, ns = info.num_cores, info.num_subcores
    nw = nc * ns
    rows_per_w = N // nw
    mesh = plsc.VectorSubcoreMesh(core_axis_name="c", subcore_axis_name="s")
    return pl.kernel(
        functools.partial(_scatter_body, rows_per_w=rows_per_w, nc=nc),
        out_type=jax.ShapeDtypeStruct((2 * N, d), jnp.float32),
        mesh=mesh,
        scratch_types=[
            pltpu.VMEM((_NBUF, _CH, d), jnp.float32),
            pltpu.VMEM((_NBUF, _CH), jnp.int32),
            pltpu.VMEM((_NBUF, _CH), jnp.int32),
            pltpu.SemaphoreType.DMA,
            pltpu.SemaphoreType.DMA,
            pltpu.SemaphoreType.DMA,
            pltpu.SemaphoreType.DMA,
            pltpu.SemaphoreType.DMA,
            pltpu.SemaphoreType.DMA,
        ],
    )(x2, pos0, pos1)


def kernel(x, partition_indices):
    B, S, d = x.shape
    k = partition_indices.shape[2]
    n = S * k

    parts3 = partition_indices.astype(jnp.int32).reshape(B, 128, n // 128)
    inv, p0, p1, cs = _compute_positions(parts3)

    inverse = inv.reshape(B, S, k)
    cu_seqlens = jnp.concatenate(
        [jnp.zeros((1,), jnp.int32), cs.reshape(-1)])

    x2 = x.reshape(B * S, d)
    pos0 = p0.reshape(-1)
    pos1 = p1.reshape(-1)
    packed = _scatter_rows(x2, pos0, pos1)
    return packed, cu_seqlens, inverse


# speedup vs baseline: 1.1338x; 1.1338x over previous
import functools

import jax
import jax.numpy as jnp
from jax import lax
from jax.experimental import pallas as pl
from jax.experimental.pallas import tpu as pltpu
from jax.experimental.pallas import tpu_sc as plsc

_P = 64


def _positions_kernel(parts_ref, inv_ref, cs_ref, *, n):
    b = pl.program_id(0)
    p2 = parts_ref[0]
    R, C = p2.shape
    Q = R * _P

    f32 = jnp.float32
    hi = lax.Precision.HIGHEST

    qi_qc = lax.broadcasted_iota(jnp.int32, (Q, C), 0)
    piot = (qi_qc % _P)
    oh = (lax.broadcasted_iota(jnp.int32, (Q, _P), 0) % _P
          == lax.broadcasted_iota(jnp.int32, (Q, _P), 1)).astype(f32)
    ci = lax.broadcasted_iota(jnp.int32, (C, C), 0)
    cj = lax.broadcasted_iota(jnp.int32, (C, C), 1)
    u_strict = (ci < cj).astype(f32)
    l_strict = (cj < ci).astype(f32)
    p_i = lax.broadcasted_iota(jnp.int32, (_P, _P), 0)
    p_j = lax.broadcasted_iota(jnp.int32, (_P, _P), 1)
    up_strict = (p_i < p_j).astype(f32)

    rep = jnp.repeat(p2.astype(f32), _P, axis=0)
    ew = (rep == piot.astype(f32)).astype(f32)

    prefw = jnp.dot(ew, u_strict, preferred_element_type=f32)

    rc = jnp.sum(ew, axis=1, keepdims=True)
    rc2 = jnp.sum((rc * oh).reshape(R, _P, _P), axis=1)

    before2 = jnp.dot(l_strict[:, :R], rc2, preferred_element_type=f32)
    counts = jnp.sum(rc2, axis=0, keepdims=True)
    base = jnp.dot(counts, up_strict, preferred_element_type=f32,
                   precision=hi)
    bb2 = before2 + base

    bbrep = jnp.repeat(bb2, _P, axis=0)
    bbflat = jnp.sum(bbrep * oh, axis=1, keepdims=True)

    pos = jnp.sum((ew * (prefw + bbflat)).reshape(R, _P, C), axis=1)

    incl = counts + base
    inv_ref[0] = pos.astype(jnp.int32) + b * n
    cs_ref[0] = incl.astype(jnp.int32) + b * n


def _compute_positions(parts3):
    B = parts3.shape[0]
    n = parts3.shape[1] * parts3.shape[2]
    return pl.pallas_call(
        functools.partial(_positions_kernel, n=n),
        grid=(B,),
        in_specs=[pl.BlockSpec((1, 128, 128), lambda b: (b, 0, 0))],
        out_specs=[
            pl.BlockSpec((1, 128, 128), lambda b: (b, 0, 0)),
            pl.BlockSpec((1, 1, _P), lambda b: (b, 0, 0)),
        ],
        out_shape=[
            jax.ShapeDtypeStruct((B, 128, 128), jnp.int32),
            jax.ShapeDtypeStruct((B, 1, _P), jnp.int32),
        ],
    )(parts3)


_CH = 32


_NBUF = 3


def _scatter_body(x_hbm, pos0_hbm, pos1_hbm, out_hbm,
                  rows_v, idx0_v, idx1_v, l0, l1, l2, s0, s1, s2,
                  *, rows_per_w, nc):
    wid = lax.axis_index("s") * nc + lax.axis_index("c")
    w0 = wid * rows_per_w
    n_chunks = rows_per_w // _CH
    lsem = (l0, l1, l2)
    ssem = (s0, s1, s2)

    def start_load(c):
        b = c % _NBUF
        sl = pl.ds(w0 + c * _CH, _CH)
        return (
            pltpu.async_copy(pos0_hbm.at[sl], idx0_v.at[b], lsem[b]),
            pltpu.async_copy(pos1_hbm.at[sl], idx1_v.at[b], lsem[b]),
            pltpu.async_copy(x_hbm.at[sl], rows_v.at[b], lsem[b]),
        )

    def start_scatter(c):
        b = c % _NBUF
        return (
            pltpu.async_copy(rows_v.at[b], out_hbm.at[idx0_v.at[b]], ssem[b]),
            pltpu.async_copy(rows_v.at[b], out_hbm.at[idx1_v.at[b]], ssem[b]),
        )

    loads = {0: start_load(0), 1: start_load(1)}
    scats = {}
    for c in range(n_chunks):
        for h in loads.pop(c):
            h.wait()
        scats[c] = start_scatter(c)
        nxt = c + 2
        if nxt < n_chunks:
            if nxt - _NBUF in scats:
                for h in scats.pop(nxt - _NBUF):
                    h.wait()
            loads[nxt] = start_load(nxt)
    for hs in scats.values():
        for h in hs:
            h.wait()


def _scatter_rows(x2, pos0, pos1):
    N, d = x2.shape
    info = plsc.get_sparse_core_info()
    nc, ns = info.num_cores, info.num_subcores
    nw = nc * ns
    rows_per_w = N // nw
    mesh = plsc.VectorSubcoreMesh(core_axis_name="c", subcore_axis_name="s")
    return pl.kernel(
        functools.partial(_scatter_body, rows_per_w=rows_per_w, nc=nc),
        out_type=jax.ShapeDtypeStruct((2 * N, d), jnp.float32),
        mesh=mesh,
        scratch_types=[
            pltpu.VMEM((_NBUF, _CH, d), jnp.float32),
            pltpu.VMEM((_NBUF, _CH), jnp.int32),
            pltpu.VMEM((_NBUF, _CH), jnp.int32),
            pltpu.SemaphoreType.DMA,
            pltpu.SemaphoreType.DMA,
            pltpu.SemaphoreType.DMA,
            pltpu.SemaphoreType.DMA,
            pltpu.SemaphoreType.DMA,
            pltpu.SemaphoreType.DMA,
        ],
    )(x2, pos0, pos1)


def kernel(x, partition_indices):
    B, S, d = x.shape
    k = partition_indices.shape[2]
    n = S * k

    parts3 = partition_indices.astype(jnp.int32).reshape(B, 128, n // 128)
    inv, cs = _compute_positions(parts3)

    inverse = inv.reshape(B, S, k)
    cu_seqlens = jnp.concatenate(
        [jnp.zeros((1,), jnp.int32), cs.reshape(-1)])

    x2 = x.reshape(B * S, d)
    pos0 = inv[:, :, 0::2].reshape(-1)
    pos1 = inv[:, :, 1::2].reshape(-1)
    packed = _scatter_rows(x2, pos0, pos1)
    return packed, cu_seqlens, inverse


# speedup vs baseline: 1.1572x; 1.0207x over previous
import functools

import jax
import jax.numpy as jnp
from jax import lax
from jax.experimental import pallas as pl
from jax.experimental.pallas import tpu as pltpu
from jax.experimental.pallas import tpu_sc as plsc

_P = 64


def _positions_kernel(parts_ref, inv_ref, cs_ref, *, n):
    b = pl.program_id(0)
    pair = parts_ref[0]
    R = 128
    C = 128
    Q = R * _P

    f32_ = jnp.float32
    oh64 = (lax.broadcasted_iota(jnp.int32, (Q, _P), 0) % _P
            == lax.broadcasted_iota(jnp.int32, (Q, _P), 1)).astype(f32_)
    a0 = pair[:, 0:1].astype(f32_)
    a1 = pair[:, 1:2].astype(f32_)
    b0 = jnp.sum((a0 * oh64).reshape(R, _P, _P), axis=1)
    b1 = jnp.sum((a1 * oh64).reshape(R, _P, _P), axis=1)
    c_iota = lax.broadcasted_iota(jnp.int32, (_P, C), 1)
    cc_iota = lax.broadcasted_iota(jnp.int32, (_P, C), 0)
    pe = ((c_iota // 2 == cc_iota)
          & (c_iota % 2 == 0)).astype(f32_)
    po = ((c_iota // 2 == cc_iota)
          & (c_iota % 2 == 1)).astype(f32_)
    p2 = (jnp.dot(b0, pe, preferred_element_type=f32_)
          + jnp.dot(b1, po, preferred_element_type=f32_)).astype(jnp.int32)

    f32 = jnp.float32
    hi = lax.Precision.HIGHEST

    qi_qc = lax.broadcasted_iota(jnp.int32, (Q, C), 0)
    piot = (qi_qc % _P)
    oh = (lax.broadcasted_iota(jnp.int32, (Q, _P), 0) % _P
          == lax.broadcasted_iota(jnp.int32, (Q, _P), 1)).astype(f32)
    ci = lax.broadcasted_iota(jnp.int32, (C, C), 0)
    cj = lax.broadcasted_iota(jnp.int32, (C, C), 1)
    u_strict = (ci < cj).astype(f32)
    l_strict = (cj < ci).astype(f32)
    p_i = lax.broadcasted_iota(jnp.int32, (_P, _P), 0)
    p_j = lax.broadcasted_iota(jnp.int32, (_P, _P), 1)
    up_strict = (p_i < p_j).astype(f32)

    rep = jnp.repeat(p2.astype(f32), _P, axis=0)
    ew = (rep == piot.astype(f32)).astype(f32)

    prefw = jnp.dot(ew, u_strict, preferred_element_type=f32)

    rc = jnp.sum(ew, axis=1, keepdims=True)
    rc2 = jnp.sum((rc * oh).reshape(R, _P, _P), axis=1)

    before2 = jnp.dot(l_strict[:, :R], rc2, preferred_element_type=f32)
    counts = jnp.sum(rc2, axis=0, keepdims=True)
    base = jnp.dot(counts, up_strict, preferred_element_type=f32,
                   precision=hi)
    bb2 = before2 + base

    bbrep = jnp.repeat(bb2, _P, axis=0)
    bbflat = jnp.sum(bbrep * oh, axis=1, keepdims=True)

    pos = jnp.sum((ew * (prefw + bbflat)).reshape(R, _P, C), axis=1)

    incl = counts + base
    inv_ref[0] = pos.astype(jnp.int32) + b * n
    cs_ref[0] = incl.astype(jnp.int32) + b * n


def _compute_positions(parts3):
    B = parts3.shape[0]
    n = parts3.shape[1] * parts3.shape[2]
    return pl.pallas_call(
        functools.partial(_positions_kernel, n=n),
        grid=(B,),
        in_specs=[pl.BlockSpec((1, n // 2, 2), lambda b: (b, 0, 0))],
        out_specs=[
            pl.BlockSpec((1, 128, 128), lambda b: (b, 0, 0)),
            pl.BlockSpec((1, 1, _P), lambda b: (b, 0, 0)),
        ],
        out_shape=[
            jax.ShapeDtypeStruct((B, 128, 128), jnp.int32),
            jax.ShapeDtypeStruct((B, 1, _P), jnp.int32),
        ],
    )(parts3)


_CH = 32


_NBUF = 3


def _scatter_body(x_hbm, pos0_hbm, pos1_hbm, out_hbm,
                  rows_v, idx0_v, idx1_v, l0, l1, l2, s0, s1, s2,
                  *, rows_per_w, nc):
    wid = lax.axis_index("s") * nc + lax.axis_index("c")
    w0 = wid * rows_per_w
    n_chunks = rows_per_w // _CH
    lsem = (l0, l1, l2)
    ssem = (s0, s1, s2)

    def start_load(c):
        b = c % _NBUF
        sl = pl.ds(w0 + c * _CH, _CH)
        return (
            pltpu.async_copy(pos0_hbm.at[sl], idx0_v.at[b], lsem[b]),
            pltpu.async_copy(pos1_hbm.at[sl], idx1_v.at[b], lsem[b]),
            pltpu.async_copy(x_hbm.at[sl], rows_v.at[b], lsem[b]),
        )

    def start_scatter(c):
        b = c % _NBUF
        return (
            pltpu.async_copy(rows_v.at[b], out_hbm.at[idx0_v.at[b]], ssem[b]),
            pltpu.async_copy(rows_v.at[b], out_hbm.at[idx1_v.at[b]], ssem[b]),
        )

    loads = {0: start_load(0), 1: start_load(1)}
    scats = {}
    for c in range(n_chunks):
        for h in loads.pop(c):
            h.wait()
        scats[c] = start_scatter(c)
        nxt = c + 2
        if nxt < n_chunks:
            if nxt - _NBUF in scats:
                for h in scats.pop(nxt - _NBUF):
                    h.wait()
            loads[nxt] = start_load(nxt)
    for hs in scats.values():
        for h in hs:
            h.wait()


def _scatter_rows(x2, pos0, pos1):
    N, d = x2.shape
    info = plsc.get_sparse_core_info()
    nc, ns = info.num_cores, info.num_subcores
    nw = nc * ns
    rows_per_w = N // nw
    mesh = plsc.VectorSubcoreMesh(core_axis_name="c", subcore_axis_name="s")
    return pl.kernel(
        functools.partial(_scatter_body, rows_per_w=rows_per_w, nc=nc),
        out_type=jax.ShapeDtypeStruct((2 * N, d), jnp.float32),
        mesh=mesh,
        scratch_types=[
            pltpu.VMEM((_NBUF, _CH, d), jnp.float32),
            pltpu.VMEM((_NBUF, _CH), jnp.int32),
            pltpu.VMEM((_NBUF, _CH), jnp.int32),
            pltpu.SemaphoreType.DMA,
            pltpu.SemaphoreType.DMA,
            pltpu.SemaphoreType.DMA,
            pltpu.SemaphoreType.DMA,
            pltpu.SemaphoreType.DMA,
            pltpu.SemaphoreType.DMA,
        ],
    )(x2, pos0, pos1)


def kernel(x, partition_indices):
    B, S, d = x.shape
    k = partition_indices.shape[2]
    n = S * k

    inv, cs = _compute_positions(partition_indices.astype(jnp.int32))

    inverse = inv.reshape(B, S, k)
    cu_seqlens = jnp.concatenate(
        [jnp.zeros((1,), jnp.int32), cs.reshape(-1)])

    x2 = x.reshape(B * S, d)
    pos0 = inv[:, :, 0::2].reshape(-1)
    pos1 = inv[:, :, 1::2].reshape(-1)
    packed = _scatter_rows(x2, pos0, pos1)
    return packed, cu_seqlens, inverse
